# baseline (speedup 1.0000x reference)
import functools

import jax
import jax.numpy as jnp
from jax import lax
from jax.experimental import pallas as pl
from jax.experimental.pallas import tpu as pltpu
from jax.experimental.pallas import tpu_sc as plsc

_B = 128
_D = 512
_N = _B * _D
_K = 2048
_NC = 2
_NS = 16
_L = 16
_NW = _NC * _NS
_EPW = _N // _NW
_CPS = _K // _NS
_NVR = _CPS // _L


@functools.partial(
    pl.kernel,
    out_type=(
        jax.ShapeDtypeStruct((_N,), jnp.float32),
        jax.ShapeDtypeStruct((_NW, _L), jnp.float32),
    ),
    mesh=plsc.VectorSubcoreMesh(core_axis_name="c", subcore_axis_name="s",
                                num_cores=_NC, num_subcores=_NS),
    compiler_params=pltpu.CompilerParams(needs_layout_passes=False),
    scratch_types=[
        pltpu.VMEM((_K,), jnp.float32),
        pltpu.VMEM((_CPS,), jnp.int32),
        pltpu.VMEM_SHARED((_K,), jnp.int32),
        pltpu.VMEM((_K,), jnp.int32),
        pltpu.VMEM((_K,), jnp.float32),
        pltpu.VMEM((_EPW,), jnp.float32),
        pltpu.VMEM((_EPW,), jnp.float32),
        pltpu.VMEM((_L,), jnp.float32),
    ],
)
def _vq_snap(x_hbm, emb_hbm, out_hbm, sq_hbm,
             emb_v, ranks_v, shr_ranks, ranks_all, sorted_v, x_v, o_v, acc_v):
    cid = lax.axis_index("c")
    sid = lax.axis_index("s")
    wid = sid * _NC + cid

    pltpu.sync_copy(emb_hbm, emb_v)
    i0 = sid * _CPS
    lanes = lax.iota(jnp.int32, _L)
    civ = [emb_v[pl.ds(i0 + r * _L, _L)] for r in range(_NVR)]
    iiv = [i0 + r * _L + lanes for r in range(_NVR)]
    one = jnp.ones((_L,), jnp.int32)
    zero = jnp.zeros((_L,), jnp.int32)

    jv0 = sid * _NVR

    def cnt_below(jv, cnts):
        cjv = emb_v[pl.ds(jv * _L, _L)]
        for lane in range(_L):
            cj = jnp.full((_L,), cjv[lane])
            cnts = tuple(c + jnp.where(cj <= cv, one, zero)
                         for c, cv in zip(cnts, civ))
        return cnts

    def cnt_mid(jv, cnts):
        cjv = emb_v[pl.ds(jv * _L, _L)]
        for lane in range(_L):
            cj = jnp.full((_L,), cjv[lane])
            j = jv * _L + lane
            out = []
            for c, cv, iv in zip(cnts, civ, iiv):
                hit = jnp.where(j < iv, cj <= cv, cj < cv)
                out.append(c + jnp.where(hit, one, zero))
            cnts = tuple(out)
        return cnts

    def cnt_above(jv, cnts):
        cjv = emb_v[pl.ds(jv * _L, _L)]
        for lane in range(_L):
            cj = jnp.full((_L,), cjv[lane])
            cnts = tuple(c + jnp.where(cj < cv, one, zero)
                         for c, cv in zip(cnts, civ))
        return cnts

    with jax.named_scope("rank_phase"):
        cnts = tuple(zero for _ in range(_NVR))
        cnts = lax.fori_loop(0, jv0, cnt_below, cnts)
        cnts = lax.fori_loop(jv0, jv0 + _NVR, cnt_mid, cnts)
        cnts = lax.fori_loop(jv0 + _NVR, _K // _L, cnt_above, cnts)
        for r in range(_NVR):
            ranks_v[pl.ds(r * _L, _L)] = cnts[r]

    with jax.named_scope("sort_build"):
        pltpu.sync_copy(ranks_v, shr_ranks.at[pl.ds(i0, _CPS)])
        plsc.subcore_barrier()
        pltpu.sync_copy(shr_ranks, ranks_all)

        def scat(jv, carry):
            v = emb_v[pl.ds(jv * _L, _L)]
            r = ranks_all[pl.ds(jv * _L, _L)]
            plsc.store_scatter(sorted_v, [r], v)
            return carry

        lax.fori_loop(0, _K // _L, scat, 0)

    base = wid * _EPW
    with jax.named_scope("x_dma"):
        pltpu.sync_copy(x_hbm.at[pl.ds(base, _EPW)], x_v)

    def search(v, acc):
        xv = x_v[pl.ds(v * _L, _L)]
        pos = jnp.zeros((_L,), jnp.int32)
        step = _K // 2
        while step >= 1:
            c = plsc.load_gather(sorted_v, [pos + (step - 1)])
            pos = jnp.where(c < xv, pos + step, pos)
            step //= 2
        i1 = jnp.maximum(pos - 1, 0)
        i2 = jnp.minimum(pos, _K - 1)
        a = plsc.load_gather(sorted_v, [i1])
        b = plsc.load_gather(sorted_v, [i2])
        da = (a - xv) * (a - xv)
        db = (b - xv) * (b - xv)
        lc = jnp.where(db < da, b, a)
        o_v[pl.ds(v * _L, _L)] = xv + (lc - xv)
        d = lc - xv
        return acc + d * d

    with jax.named_scope("search_phase"):
        acc = lax.fori_loop(0, _EPW // _L, search,
                            jnp.zeros((_L,), jnp.float32))
    acc_v[...] = acc
    pltpu.sync_copy(o_v, out_hbm.at[pl.ds(base, _EPW)])
    pltpu.sync_copy(acc_v, sq_hbm.at[wid])


def kernel(encoded, embeddings):
    x = encoded.reshape(-1)
    emb = embeddings.reshape(-1)
    out, sq = _vq_snap(x, emb)
    latent_code_st = out.reshape(encoded.shape)
    loss = 2.0 * (jnp.sum(sq) / encoded.shape[0])
    return latent_code_st, loss


# speedup vs baseline: 1.1048x; 1.1048x over previous
import functools

import jax
import jax.numpy as jnp
from jax import lax
from jax.experimental import pallas as pl
from jax.experimental.pallas import tpu as pltpu
from jax.experimental.pallas import tpu_sc as plsc

_B = 128
_D = 512
_N = _B * _D
_K = 2048
_NC = 2
_NS = 16
_L = 16
_NW = _NC * _NS
_EPW = _N // _NW
_CPS = _K // _NS
_NVR = _CPS // _L


@functools.partial(
    pl.kernel,
    out_type=(
        jax.ShapeDtypeStruct((_N,), jnp.float32),
        jax.ShapeDtypeStruct((_NW, _L), jnp.float32),
    ),
    mesh=plsc.VectorSubcoreMesh(core_axis_name="c", subcore_axis_name="s",
                                num_cores=_NC, num_subcores=_NS),
    compiler_params=pltpu.CompilerParams(needs_layout_passes=False),
    scratch_types=[
        pltpu.VMEM((_K,), jnp.float32),
        pltpu.VMEM((_CPS,), jnp.int32),
        pltpu.VMEM_SHARED((_K,), jnp.int32),
        pltpu.VMEM((_K,), jnp.int32),
        pltpu.VMEM((_K,), jnp.float32),
        pltpu.VMEM((_EPW,), jnp.float32),
        pltpu.VMEM((_EPW,), jnp.float32),
        pltpu.VMEM((_L,), jnp.float32),
    ],
)
def _vq_snap(x_hbm, emb_hbm, out_hbm, sq_hbm,
             emb_v, ranks_v, shr_ranks, ranks_all, sorted_v, x_v, o_v, acc_v):
    cid = lax.axis_index("c")
    sid = lax.axis_index("s")
    wid = sid * _NC + cid

    pltpu.sync_copy(emb_hbm, emb_v)
    i0 = sid * _CPS
    lanes = lax.iota(jnp.int32, _L)
    civ = [emb_v[pl.ds(i0 + r * _L, _L)] for r in range(_NVR)]
    iiv = [i0 + r * _L + lanes for r in range(_NVR)]
    one = jnp.ones((_L,), jnp.float32)
    zero = jnp.zeros((_L,), jnp.float32)

    jv0 = sid * _NVR

    def cnt_below(jv, cnts):
        cjv = emb_v[pl.ds(jv * _L, _L)]
        for lane in range(_L):
            cj = jnp.full((_L,), cjv[lane])
            cnts = tuple(c + jnp.where(cj <= cv, one, zero)
                         for c, cv in zip(cnts, civ))
        return cnts

    def cnt_mid(jv, cnts):
        cjv = emb_v[pl.ds(jv * _L, _L)]
        for lane in range(_L):
            cj = jnp.full((_L,), cjv[lane])
            j = jv * _L + lane
            out = []
            for c, cv, iv in zip(cnts, civ, iiv):
                hit = jnp.where(j < iv, cj <= cv, cj < cv)
                out.append(c + jnp.where(hit, one, zero))
            cnts = tuple(out)
        return cnts

    def cnt_above(jv, cnts):
        cjv = emb_v[pl.ds(jv * _L, _L)]
        for lane in range(_L):
            cj = jnp.full((_L,), cjv[lane])
            cnts = tuple(c + jnp.where(cj < cv, one, zero)
                         for c, cv in zip(cnts, civ))
        return cnts

    with jax.named_scope("rank_phase"):
        cnts = tuple(zero for _ in range(_NVR))
        cnts = lax.fori_loop(0, jv0, cnt_below, cnts)
        cnts = lax.fori_loop(jv0, jv0 + _NVR, cnt_mid, cnts)
        cnts = lax.fori_loop(jv0 + _NVR, _K // _L, cnt_above, cnts)
        for r in range(_NVR):
            ranks_v[pl.ds(r * _L, _L)] = cnts[r].astype(jnp.int32)

    with jax.named_scope("sort_build"):
        pltpu.sync_copy(ranks_v, shr_ranks.at[pl.ds(i0, _CPS)])
        plsc.subcore_barrier()
        pltpu.sync_copy(shr_ranks, ranks_all)

        def scat(jv, carry):
            v = emb_v[pl.ds(jv * _L, _L)]
            r = ranks_all[pl.ds(jv * _L, _L)]
            plsc.store_scatter(sorted_v, [r], v)
            return carry

        lax.fori_loop(0, _K // _L, scat, 0)

    base = wid * _EPW
    with jax.named_scope("x_dma"):
        pltpu.sync_copy(x_hbm.at[pl.ds(base, _EPW)], x_v)

    _UNROLL = 4

    def snap_one(xv):
        pos = jnp.zeros((_L,), jnp.int32)
        step = _K // 2
        while step >= 1:
            c = plsc.load_gather(sorted_v, [pos + (step - 1)])
            pos = jnp.where(c < xv, pos + step, pos)
            step //= 2
        i1 = jnp.maximum(pos - 1, 0)
        i2 = jnp.minimum(pos, _K - 1)
        a = plsc.load_gather(sorted_v, [i1])
        b = plsc.load_gather(sorted_v, [i2])
        da = (a - xv) * (a - xv)
        db = (b - xv) * (b - xv)
        return jnp.where(db < da, b, a)

    def search(g, accs):
        new_accs = []
        for u in range(_UNROLL):
            v = g * _UNROLL + u
            xv = x_v[pl.ds(v * _L, _L)]
            lc = snap_one(xv)
            o_v[pl.ds(v * _L, _L)] = xv + (lc - xv)
            d = lc - xv
            new_accs.append(accs[u] + d * d)
        return tuple(new_accs)

    with jax.named_scope("search_phase"):
        accs = lax.fori_loop(0, _EPW // _L // _UNROLL, search,
                             tuple(jnp.zeros((_L,), jnp.float32)
                                   for _ in range(_UNROLL)))
        acc = accs[0]
        for u in range(1, _UNROLL):
            acc = acc + accs[u]
    acc_v[...] = acc
    pltpu.sync_copy(o_v, out_hbm.at[pl.ds(base, _EPW)])
    pltpu.sync_copy(acc_v, sq_hbm.at[wid])


def kernel(encoded, embeddings):
    x = encoded.reshape(-1)
    emb = embeddings.reshape(-1)
    out, sq = _vq_snap(x, emb)
    latent_code_st = out.reshape(encoded.shape)
    loss = 2.0 * (jnp.sum(sq) / encoded.shape[0])
    return latent_code_st, loss


# speedup vs baseline: 1.3351x; 1.2085x over previous
import functools

import jax
import jax.numpy as jnp
from jax import lax
from jax.experimental import pallas as pl
from jax.experimental.pallas import tpu as pltpu
from jax.experimental.pallas import tpu_sc as plsc

_B = 128
_D = 512
_N = _B * _D
_K = 2048
_NC = 2
_NS = 16
_L = 16
_NW = _NC * _NS
_EPW = _N // _NW
_CPS = _K // _NS
_NVR = _CPS // _L


@functools.partial(
    pl.kernel,
    out_type=(
        jax.ShapeDtypeStruct((_N,), jnp.float32),
        jax.ShapeDtypeStruct((_NW, _L), jnp.float32),
    ),
    mesh=plsc.VectorSubcoreMesh(core_axis_name="c", subcore_axis_name="s",
                                num_cores=_NC, num_subcores=_NS),
    compiler_params=pltpu.CompilerParams(needs_layout_passes=False),
    scratch_types=[
        pltpu.VMEM((_K,), jnp.float32),
        pltpu.VMEM((_CPS,), jnp.int32),
        pltpu.VMEM_SHARED((_K,), jnp.int32),
        pltpu.VMEM((_K,), jnp.int32),
        pltpu.VMEM((_K,), jnp.float32),
        pltpu.VMEM((_EPW,), jnp.float32),
        pltpu.VMEM((_EPW,), jnp.float32),
        pltpu.VMEM((_L,), jnp.float32),
    ],
)
def _vq_snap(x_hbm, emb_hbm, out_hbm, sq_hbm,
             emb_v, ranks_v, shr_ranks, ranks_all, sorted_v, x_v, o_v, acc_v):
    cid = lax.axis_index("c")
    sid = lax.axis_index("s")
    wid = sid * _NC + cid

    pltpu.sync_copy(emb_hbm, emb_v)
    i0 = sid * _CPS
    lanes = lax.iota(jnp.int32, _L)
    civ = [emb_v[pl.ds(i0 + r * _L, _L)] for r in range(_NVR)]
    iiv = [i0 + r * _L + lanes for r in range(_NVR)]

    jv0 = sid * _NVR

    def _sign(x):
        return lax.shift_right_logical(plsc.bitcast(x, jnp.int32), 31)

    def cnt_below(jv, accs):
        cjv = emb_v[pl.ds(jv * _L, _L)]
        for lane in range(_L):
            cj = jnp.full((_L,), cjv[lane])
            accs = tuple(a + _sign(cv - cj) for a, cv in zip(accs, civ))
        return accs

    def cnt_mid(jv, accs):
        cjv = emb_v[pl.ds(jv * _L, _L)]
        onei = jnp.ones((_L,), jnp.int32)
        for lane in range(_L):
            cj = jnp.full((_L,), cjv[lane])
            j = jv * _L + lane
            out = []
            for a, cv, iv in zip(accs, civ, iiv):
                ltv = _sign(cj - cv)
                gtv = _sign(cv - cj)
                eq = onei - ltv - gtv
                jlt = lax.shift_right_logical(j - iv, 31)
                out.append(a - (ltv + (eq & jlt)))
            accs = tuple(out)
        return accs

    def cnt_above(jv, accs):
        cjv = emb_v[pl.ds(jv * _L, _L)]
        for lane in range(_L):
            cj = jnp.full((_L,), cjv[lane])
            accs = tuple(a - _sign(cj - cv) for a, cv in zip(accs, civ))
        return accs

    with jax.named_scope("rank_phase"):
        zeroi = jnp.zeros((_L,), jnp.int32)
        accs = tuple(zeroi for _ in range(_NVR))
        accs = lax.fori_loop(0, jv0, cnt_below, accs)
        accs = lax.fori_loop(jv0, jv0 + _NVR, cnt_mid, accs)
        accs = lax.fori_loop(jv0 + _NVR, _K // _L, cnt_above, accs)
        n_below = jv0 * _L
        for r in range(_NVR):
            ranks_v[pl.ds(r * _L, _L)] = n_below - accs[r]

    with jax.named_scope("sort_build"):
        pltpu.sync_copy(ranks_v, shr_ranks.at[pl.ds(i0, _CPS)])
        plsc.subcore_barrier()
        pltpu.sync_copy(shr_ranks, ranks_all)

        def scat(jv, carry):
            v = emb_v[pl.ds(jv * _L, _L)]
            r = ranks_all[pl.ds(jv * _L, _L)]
            plsc.store_scatter(sorted_v, [r], v)
            return carry

        lax.fori_loop(0, _K // _L, scat, 0)

    base = wid * _EPW
    with jax.named_scope("x_dma"):
        pltpu.sync_copy(x_hbm.at[pl.ds(base, _EPW)], x_v)

    def snap_one(xv):
        pos = jnp.zeros((_L,), jnp.int32)
        step = _K // 2
        while step >= 1:
            c = plsc.load_gather(sorted_v, [pos + (step - 1)])
            pos = jnp.where(c < xv, pos + step, pos)
            step //= 2
        i1 = jnp.maximum(pos - 1, 0)
        i2 = jnp.minimum(pos, _K - 1)
        a = plsc.load_gather(sorted_v, [i1])
        b = plsc.load_gather(sorted_v, [i2])
        da = (a - xv) * (a - xv)
        db = (b - xv) * (b - xv)
        return jnp.where(db < da, b, a)

    with jax.named_scope("search_phase"):
        @plsc.parallel_loop(0, _EPW // _L, unroll=4,
                            carry=jnp.zeros((_L,), jnp.float32))
        def acc(v, a):
            xv = x_v[pl.ds(v * _L, _L)]
            lc = snap_one(xv)
            o_v[pl.ds(v * _L, _L)] = xv + (lc - xv)
            d = lc - xv
            return a + d * d
    acc_v[...] = acc
    pltpu.sync_copy(o_v, out_hbm.at[pl.ds(base, _EPW)])
    pltpu.sync_copy(acc_v, sq_hbm.at[wid])


def kernel(encoded, embeddings):
    x = encoded.reshape(-1)
    emb = embeddings.reshape(-1)
    out, sq = _vq_snap(x, emb)
    latent_code_st = out.reshape(encoded.shape)
    loss = 2.0 * (jnp.sum(sq) / encoded.shape[0])
    return latent_code_st, loss


# speedup vs baseline: 2.0344x; 1.5238x over previous
import functools

import jax
import jax.numpy as jnp
from jax import lax
from jax.experimental import pallas as pl
from jax.experimental.pallas import tpu as pltpu
from jax.experimental.pallas import tpu_sc as plsc

_B = 128
_D = 512
_N = _B * _D
_K = 2048
_NC = 2
_NS = 16
_L = 16
_NW = _NC * _NS
_EPW = _N // _NW
_CPS = _K // _NS
_NVR = _CPS // _L


@functools.partial(
    pl.kernel,
    out_type=(
        jax.ShapeDtypeStruct((_N,), jnp.float32),
        jax.ShapeDtypeStruct((_NW, _L), jnp.float32),
    ),
    mesh=plsc.VectorSubcoreMesh(core_axis_name="c", subcore_axis_name="s",
                                num_cores=_NC, num_subcores=_NS),
    compiler_params=pltpu.CompilerParams(needs_layout_passes=False),
    scratch_types=[
        pltpu.VMEM((_K,), jnp.float32),
        pltpu.VMEM((_CPS,), jnp.int32),
        pltpu.VMEM_SHARED((_K,), jnp.int32),
        pltpu.VMEM((_K,), jnp.int32),
        pltpu.VMEM((_K,), jnp.float32),
        pltpu.VMEM((_EPW,), jnp.float32),
        pltpu.VMEM((_EPW,), jnp.float32),
        pltpu.VMEM((_L,), jnp.float32),
    ],
)
def _vq_snap(x_hbm, emb_hbm, out_hbm, sq_hbm,
             emb_v, ranks_v, shr_ranks, ranks_all, sorted_v, x_v, o_v, acc_v):
    cid = lax.axis_index("c")
    sid = lax.axis_index("s")
    wid = sid * _NC + cid

    pltpu.sync_copy(emb_hbm, emb_v)
    i0 = sid * _CPS
    lanes = lax.iota(jnp.int32, _L)
    civ = [emb_v[pl.ds(i0 + r * _L, _L)] for r in range(_NVR)]
    iiv = [i0 + r * _L + lanes for r in range(_NVR)]

    jv0 = sid * _NVR

    def _sign(x):
        return lax.shift_right_logical(plsc.bitcast(x, jnp.int32), 31)

    _H = _NVR // 2

    def make_bodies(civ_h, iiv_h, lane_lo):
        def cnt_below(jv, accs):
            cjv = emb_v[pl.ds(jv * _L, _L)]
            for lane in range(lane_lo, lane_lo + 8):
                cj = jnp.full((_L,), cjv[lane])
                accs = tuple(a + _sign(cv - cj)
                             for a, cv in zip(accs, civ_h))
            return accs

        def cnt_mid(jv, accs):
            cjv = emb_v[pl.ds(jv * _L, _L)]
            onei = jnp.ones((_L,), jnp.int32)
            for lane in range(lane_lo, lane_lo + 8):
                cj = jnp.full((_L,), cjv[lane])
                j = jv * _L + lane
                out = []
                for a, cv, iv in zip(accs, civ_h, iiv_h):
                    ltv = _sign(cj - cv)
                    gtv = _sign(cv - cj)
                    eq = onei - ltv - gtv
                    jlt = lax.shift_right_logical(j - iv, 31)
                    out.append(a - (ltv + (eq & jlt)))
                accs = tuple(out)
            return accs

        def cnt_above(jv, accs):
            cjv = emb_v[pl.ds(jv * _L, _L)]
            for lane in range(lane_lo, lane_lo + 8):
                cj = jnp.full((_L,), cjv[lane])
                accs = tuple(a - _sign(cj - cv)
                             for a, cv in zip(accs, civ_h))
            return accs

        return cnt_below, cnt_mid, cnt_above

    with jax.named_scope("rank_phase"):
        zeroi = jnp.zeros((_L,), jnp.int32)
        n_below = jv0 * _L
        for half in range(2):
            civ_h = civ[half * _H:(half + 1) * _H]
            iiv_h = iiv[half * _H:(half + 1) * _H]
            accs = tuple(zeroi for _ in range(_H))
            for lane_lo in (0, 8):
                below, mid, above = make_bodies(civ_h, iiv_h, lane_lo)
                accs = lax.fori_loop(0, jv0, below, accs)
                accs = lax.fori_loop(jv0, jv0 + _NVR, mid, accs)
                accs = lax.fori_loop(jv0 + _NVR, _K // _L, above, accs)
            for r in range(_H):
                ranks_v[pl.ds((half * _H + r) * _L, _L)] = n_below - accs[r]

    with jax.named_scope("sort_build"):
        pltpu.sync_copy(ranks_v, shr_ranks.at[pl.ds(i0, _CPS)])
        plsc.subcore_barrier()
        pltpu.sync_copy(shr_ranks, ranks_all)

        def scat(jv, carry):
            v = emb_v[pl.ds(jv * _L, _L)]
            r = ranks_all[pl.ds(jv * _L, _L)]
            plsc.store_scatter(sorted_v, [r], v)
            return carry

        lax.fori_loop(0, _K // _L, scat, 0)

    base = wid * _EPW
    with jax.named_scope("x_dma"):
        pltpu.sync_copy(x_hbm.at[pl.ds(base, _EPW)], x_v)

    def snap_one(xv):
        pos = jnp.zeros((_L,), jnp.int32)
        step = _K // 2
        while step >= 1:
            c = plsc.load_gather(sorted_v, [pos + (step - 1)])
            pos = jnp.where(c < xv, pos + step, pos)
            step //= 2
        i1 = jnp.maximum(pos - 1, 0)
        i2 = jnp.minimum(pos, _K - 1)
        a = plsc.load_gather(sorted_v, [i1])
        b = plsc.load_gather(sorted_v, [i2])
        da = (a - xv) * (a - xv)
        db = (b - xv) * (b - xv)
        return jnp.where(db < da, b, a)

    with jax.named_scope("search_phase"):
        @plsc.parallel_loop(0, _EPW // _L, unroll=4,
                            carry=jnp.zeros((_L,), jnp.float32))
        def acc(v, a):
            xv = x_v[pl.ds(v * _L, _L)]
            lc = snap_one(xv)
            o_v[pl.ds(v * _L, _L)] = xv + (lc - xv)
            d = lc - xv
            return a + d * d
    acc_v[...] = acc
    pltpu.sync_copy(o_v, out_hbm.at[pl.ds(base, _EPW)])
    pltpu.sync_copy(acc_v, sq_hbm.at[wid])


def kernel(encoded, embeddings):
    x = encoded.reshape(-1)
    emb = embeddings.reshape(-1)
    out, sq = _vq_snap(x, emb)
    latent_code_st = out.reshape(encoded.shape)
    loss = 2.0 * (jnp.sum(sq) / encoded.shape[0])
    return latent_code_st, loss


# speedup vs baseline: 2.0701x; 1.0176x over previous
import functools

import jax
import jax.numpy as jnp
from jax import lax
from jax.experimental import pallas as pl
from jax.experimental.pallas import tpu as pltpu
from jax.experimental.pallas import tpu_sc as plsc

_B = 128
_D = 512
_N = _B * _D
_K = 2048
_NC = 2
_NS = 16
_L = 16
_NW = _NC * _NS
_EPW = _N // _NW
_CPS = _K // _NS
_NVR = _CPS // _L


@functools.partial(
    pl.kernel,
    out_type=(
        jax.ShapeDtypeStruct((_B, _D), jnp.float32),
        jax.ShapeDtypeStruct((_NW, _L), jnp.float32),
    ),
    mesh=plsc.VectorSubcoreMesh(core_axis_name="c", subcore_axis_name="s",
                                num_cores=_NC, num_subcores=_NS),
    compiler_params=pltpu.CompilerParams(needs_layout_passes=False),
    scratch_types=[
        pltpu.VMEM((_K,), jnp.float32),
        pltpu.VMEM((_CPS,), jnp.int32),
        pltpu.VMEM_SHARED((_K,), jnp.int32),
        pltpu.VMEM((_K,), jnp.int32),
        pltpu.VMEM((_K,), jnp.float32),
        pltpu.VMEM((_EPW // _D, _D), jnp.float32),
        pltpu.VMEM((_EPW // _D, _D), jnp.float32),
        pltpu.VMEM((_L,), jnp.float32),
    ],
)
def _vq_snap(x_hbm, emb_hbm, out_hbm, sq_hbm,
             emb_v, ranks_v, shr_ranks, ranks_all, sorted_v, x_v, o_v, acc_v):
    cid = lax.axis_index("c")
    sid = lax.axis_index("s")
    wid = sid * _NC + cid

    pltpu.sync_copy(emb_hbm.at[0], emb_v)
    i0 = sid * _CPS
    lanes = lax.iota(jnp.int32, _L)
    civ = [emb_v[pl.ds(i0 + r * _L, _L)] for r in range(_NVR)]
    iiv = [i0 + r * _L + lanes for r in range(_NVR)]

    jv0 = sid * _NVR

    def _sign(x):
        return lax.shift_right_logical(plsc.bitcast(x, jnp.int32), 31)

    _H = _NVR // 2

    def make_bodies(civ_h, iiv_h, lane_lo):
        def cnt_below(jv, accs):
            cjv = emb_v[pl.ds(jv * _L, _L)]
            for lane in range(lane_lo, lane_lo + 8):
                cj = jnp.full((_L,), cjv[lane])
                accs = tuple(a + _sign(cv - cj)
                             for a, cv in zip(accs, civ_h))
            return accs

        def cnt_mid(jv, accs):
            cjv = emb_v[pl.ds(jv * _L, _L)]
            onei = jnp.ones((_L,), jnp.int32)
            for lane in range(lane_lo, lane_lo + 8):
                cj = jnp.full((_L,), cjv[lane])
                j = jv * _L + lane
                out = []
                for a, cv, iv in zip(accs, civ_h, iiv_h):
                    ltv = _sign(cj - cv)
                    gtv = _sign(cv - cj)
                    eq = onei - ltv - gtv
                    jlt = lax.shift_right_logical(j - iv, 31)
                    out.append(a - (ltv + (eq & jlt)))
                accs = tuple(out)
            return accs

        def cnt_above(jv, accs):
            cjv = emb_v[pl.ds(jv * _L, _L)]
            for lane in range(lane_lo, lane_lo + 8):
                cj = jnp.full((_L,), cjv[lane])
                accs = tuple(a - _sign(cj - cv)
                             for a, cv in zip(accs, civ_h))
            return accs

        return cnt_below, cnt_mid, cnt_above

    with jax.named_scope("rank_phase"):
        zeroi = jnp.zeros((_L,), jnp.int32)
        n_below = jv0 * _L
        for half in range(2):
            civ_h = civ[half * _H:(half + 1) * _H]
            iiv_h = iiv[half * _H:(half + 1) * _H]
            accs = tuple(zeroi for _ in range(_H))
            for lane_lo in (0, 8):
                below, mid, above = make_bodies(civ_h, iiv_h, lane_lo)
                accs = lax.fori_loop(0, jv0, below, accs)
                accs = lax.fori_loop(jv0, jv0 + _NVR, mid, accs)
                accs = lax.fori_loop(jv0 + _NVR, _K // _L, above, accs)
            for r in range(_H):
                ranks_v[pl.ds((half * _H + r) * _L, _L)] = n_below - accs[r]

    with jax.named_scope("sort_build"):
        pltpu.sync_copy(ranks_v, shr_ranks.at[pl.ds(i0, _CPS)])
        plsc.subcore_barrier()
        pltpu.sync_copy(shr_ranks, ranks_all)

        def scat(jv, carry):
            v = emb_v[pl.ds(jv * _L, _L)]
            r = ranks_all[pl.ds(jv * _L, _L)]
            plsc.store_scatter(sorted_v, [r], v)
            return carry

        lax.fori_loop(0, _K // _L, scat, 0)

    rows = _EPW // _D
    row0 = wid * rows
    with jax.named_scope("x_dma"):
        pltpu.sync_copy(x_hbm.at[pl.ds(row0, rows)], x_v)

    def snap_one(xv):
        pos = jnp.zeros((_L,), jnp.int32)
        step = _K // 2
        while step >= 1:
            c = plsc.load_gather(sorted_v, [pos + (step - 1)])
            pos = jnp.where(c < xv, pos + step, pos)
            step //= 2
        i1 = jnp.maximum(pos - 1, 0)
        i2 = jnp.minimum(pos, _K - 1)
        a = plsc.load_gather(sorted_v, [i1])
        b = plsc.load_gather(sorted_v, [i2])
        da = (a - xv) * (a - xv)
        db = (b - xv) * (b - xv)
        return jnp.where(db < da, b, a)

    with jax.named_scope("search_phase"):
        vregs_per_row = _D // _L

        @plsc.parallel_loop(0, _EPW // _L, unroll=8,
                            carry=jnp.zeros((_L,), jnp.float32))
        def acc(v, a):
            r = v // vregs_per_row
            col = (v % vregs_per_row) * _L
            xv = x_v[r, pl.ds(col, _L)]
            lc = snap_one(xv)
            o_v[r, pl.ds(col, _L)] = xv + (lc - xv)
            d = lc - xv
            return a + d * d
    acc_v[...] = acc
    pltpu.sync_copy(o_v, out_hbm.at[pl.ds(row0, rows)])
    pltpu.sync_copy(acc_v, sq_hbm.at[wid])


def kernel(encoded, embeddings):
    latent_code_st, sq = _vq_snap(encoded, embeddings)
    loss = 2.0 * (jnp.sum(sq) / encoded.shape[0])
    return latent_code_st, loss


# speedup vs baseline: 2.2647x; 1.0940x over previous
import functools

import jax
import jax.numpy as jnp
from jax import lax
from jax.experimental import pallas as pl
from jax.experimental.pallas import tpu as pltpu
from jax.experimental.pallas import tpu_sc as plsc

_B = 128
_D = 512
_N = _B * _D
_K = 2048
_NC = 2
_NS = 16
_L = 16
_NW = _NC * _NS
_EPW = _N // _NW
_CPS = _K // _NS
_NVR = _CPS // _L


@functools.partial(
    pl.kernel,
    out_type=(
        jax.ShapeDtypeStruct((_B, _D), jnp.float32),
        jax.ShapeDtypeStruct((_NW, _L), jnp.float32),
    ),
    mesh=plsc.VectorSubcoreMesh(core_axis_name="c", subcore_axis_name="s",
                                num_cores=_NC, num_subcores=_NS),
    compiler_params=pltpu.CompilerParams(needs_layout_passes=False),
    scratch_types=[
        pltpu.VMEM((_K,), jnp.float32),
        pltpu.VMEM((_CPS,), jnp.int32),
        pltpu.VMEM_SHARED((_K,), jnp.int32),
        pltpu.VMEM((_K,), jnp.int32),
        pltpu.VMEM((_K,), jnp.float32),
        pltpu.VMEM((_EPW // _D, _D), jnp.float32),
        pltpu.VMEM((_EPW // _D, _D), jnp.float32),
        pltpu.VMEM((_L,), jnp.float32),
    ],
)
def _vq_snap(x_hbm, emb_hbm, out_hbm, sq_hbm,
             emb_v, ranks_v, shr_ranks, ranks_all, sorted_v, x_v, o_v, acc_v):
    cid = lax.axis_index("c")
    sid = lax.axis_index("s")
    wid = sid * _NC + cid

    pltpu.sync_copy(emb_hbm.at[0], emb_v)
    i0 = sid * _CPS
    lanes = lax.iota(jnp.int32, _L)
    civ = [emb_v[pl.ds(i0 + r * _L, _L)] for r in range(_NVR)]
    iiv = [i0 + r * _L + lanes for r in range(_NVR)]

    jv0 = sid * _NVR

    def _sign(x):
        return lax.shift_right_logical(plsc.bitcast(x, jnp.int32), 31)

    _H = _NVR // 2

    def make_bodies(civ_h, iiv_h, lane_lo):
        def cnt_below(jv, accs):
            cjv = emb_v[pl.ds(jv * _L, _L)]
            for lane in range(lane_lo, lane_lo + 8):
                cj = jnp.full((_L,), cjv[lane])
                accs = tuple(a + _sign(cv - cj)
                             for a, cv in zip(accs, civ_h))
            return accs

        def cnt_mid(jv, accs):
            cjv = emb_v[pl.ds(jv * _L, _L)]
            onei = jnp.ones((_L,), jnp.int32)
            for lane in range(lane_lo, lane_lo + 8):
                cj = jnp.full((_L,), cjv[lane])
                j = jv * _L + lane
                out = []
                for a, cv, iv in zip(accs, civ_h, iiv_h):
                    ltv = _sign(cj - cv)
                    gtv = _sign(cv - cj)
                    eq = onei - ltv - gtv
                    jlt = lax.shift_right_logical(j - iv, 31)
                    out.append(a - (ltv + (eq & jlt)))
                accs = tuple(out)
            return accs

        def cnt_above(jv, accs):
            cjv = emb_v[pl.ds(jv * _L, _L)]
            for lane in range(lane_lo, lane_lo + 8):
                cj = jnp.full((_L,), cjv[lane])
                accs = tuple(a - _sign(cj - cv)
                             for a, cv in zip(accs, civ_h))
            return accs

        return cnt_below, cnt_mid, cnt_above

    with jax.named_scope("rank_phase"):
        zeroi = jnp.zeros((_L,), jnp.int32)
        n_below = jv0 * _L
        for half in range(2):
            civ_h = civ[half * _H:(half + 1) * _H]
            iiv_h = iiv[half * _H:(half + 1) * _H]
            accs = tuple(zeroi for _ in range(_H))
            for lane_lo in (0, 8):
                below, mid, above = make_bodies(civ_h, iiv_h, lane_lo)
                accs = lax.fori_loop(0, jv0, below, accs)
                accs = lax.fori_loop(jv0, jv0 + _NVR, mid, accs)
                accs = lax.fori_loop(jv0 + _NVR, _K // _L, above, accs)
            for r in range(_H):
                ranks_v[pl.ds((half * _H + r) * _L, _L)] = n_below - accs[r]

    with jax.named_scope("sort_build"):
        pltpu.sync_copy(ranks_v, shr_ranks.at[pl.ds(i0, _CPS)])
        plsc.subcore_barrier()
        pltpu.sync_copy(shr_ranks, ranks_all)

        def scat(jv, carry):
            v = emb_v[pl.ds(jv * _L, _L)]
            r = ranks_all[pl.ds(jv * _L, _L)]
            plsc.store_scatter(sorted_v, [r], v)
            return carry

        lax.fori_loop(0, _K // _L, scat, 0)

    rows = _EPW // _D
    row0 = wid * rows
    with jax.named_scope("x_dma"):
        pltpu.sync_copy(x_hbm.at[pl.ds(row0, rows)], x_v)

    def snap_one(xv):
        pos = jnp.zeros((_L,), jnp.int32)
        step = _K // 2
        while step >= 1:
            c = plsc.load_gather(sorted_v, [pos + (step - 1)])
            pos = jnp.where(c < xv, pos + step, pos)
            step //= 2
        i1 = jnp.maximum(pos - 1, 0)
        i2 = jnp.minimum(pos, _K - 1)
        a = plsc.load_gather(sorted_v, [i1])
        b = plsc.load_gather(sorted_v, [i2])
        da = (a - xv) * (a - xv)
        db = (b - xv) * (b - xv)
        return jnp.where(db < da, b, a)

    with jax.named_scope("search_phase"):
        vregs_per_row = _D // _L
        smin = jnp.full((_L,), sorted_v[pl.ds(0, _L)][0])
        smax = jnp.full((_L,), sorted_v[pl.ds(_K - _L, _L)][_L - 1])

        @plsc.parallel_loop(0, _EPW // _L, unroll=2,
                            carry=jnp.zeros((_L,), jnp.float32))
        def acc(v, a):
            r = v // vregs_per_row
            col = (v % vregs_per_row) * _L
            xv = x_v[r, pl.ds(col, _L)]
            lo_m = xv <= smin
            hi_m = xv >= smax
            n_out = plsc.all_reduce_population_count(lo_m | hi_m)
            lc = lax.cond(
                n_out[0] == _L,
                lambda: jnp.where(hi_m, smax, smin),
                lambda: snap_one(xv),
            )
            o_v[r, pl.ds(col, _L)] = xv + (lc - xv)
            d = lc - xv
            return a + d * d
    acc_v[...] = acc
    pltpu.sync_copy(o_v, out_hbm.at[pl.ds(row0, rows)])
    pltpu.sync_copy(acc_v, sq_hbm.at[wid])


def kernel(encoded, embeddings):
    latent_code_st, sq = _vq_snap(encoded, embeddings)
    loss = 2.0 * (jnp.sum(sq) / encoded.shape[0])
    return latent_code_st, loss


# speedup vs baseline: 3.5826x; 1.5819x over previous
import functools

import jax
import jax.numpy as jnp
from jax import lax
from jax.experimental import pallas as pl
from jax.experimental.pallas import tpu as pltpu
from jax.experimental.pallas import tpu_sc as plsc

_B = 128
_D = 512
_N = _B * _D
_K = 2048
_NC = 2
_NS = 16
_L = 16
_NW = _NC * _NS
_EPW = _N // _NW
_ROWS = _EPW // _D


@functools.partial(
    pl.kernel,
    out_type=(
        jax.ShapeDtypeStruct((_B, _D), jnp.float32),
        jax.ShapeDtypeStruct((_NW, _L), jnp.float32),
    ),
    mesh=plsc.VectorSubcoreMesh(core_axis_name="c", subcore_axis_name="s",
                                num_cores=_NC, num_subcores=_NS),
    compiler_params=pltpu.CompilerParams(needs_layout_passes=False),
    scratch_types=[
        pltpu.VMEM((_K,), jnp.float32),
        pltpu.VMEM((_ROWS, _D), jnp.float32),
        pltpu.VMEM((_ROWS, _D), jnp.float32),
        pltpu.VMEM((_L,), jnp.float32),
    ],
)
def _vq_snap(x_hbm, emb_hbm, out_hbm, sq_hbm, emb_v, x_v, o_v, acc_v):
    cid = lax.axis_index("c")
    sid = lax.axis_index("s")
    wid = sid * _NC + cid
    lanes = lax.iota(jnp.int32, _L)

    pltpu.sync_copy(emb_hbm.at[0], emb_v)
    row0 = wid * _ROWS
    with jax.named_scope("x_dma"):
        pltpu.sync_copy(x_hbm.at[pl.ds(row0, _ROWS)], x_v)

    with jax.named_scope("minmax"):
        def mm(jv, carry):
            lo, hi = carry
            cjv = emb_v[pl.ds(jv * _L, _L)]
            return jnp.minimum(lo, cjv), jnp.maximum(hi, cjv)

        lo, hi = lax.fori_loop(
            0, _K // _L, mm,
            (jnp.full((_L,), jnp.inf, jnp.float32),
             jnp.full((_L,), -jnp.inf, jnp.float32)))
        smin = jnp.full((_L,), jnp.min(lo))
        smax = jnp.full((_L,), jnp.max(hi))

    def brute_lane(xv, l_splat, lc):
        xb = jnp.take_along_axis(xv, l_splat, axis=0)

        def scan_codes(jv, carry):
            dmin, val, idx = carry
            cjv = emb_v[pl.ds(jv * _L, _L)]
            d = (cjv - xb) * (cjv - xb)
            p = d < dmin
            jvec = jv * _L + lanes
            return (jnp.where(p, d, dmin), jnp.where(p, cjv, val),
                    jnp.where(p, jvec, idx))

        big = jnp.full((_L,), 3.4e38, jnp.float32)
        dmin, val, idx = lax.fori_loop(
            0, _K // _L, scan_codes,
            (big, jnp.zeros((_L,), jnp.float32),
             jnp.zeros((_L,), jnp.int32)))
        g = jnp.min(dmin)
        cand = jnp.where(dmin == g, idx, _K)
        bi = jnp.min(cand)
        value = jnp.max(jnp.where(cand == bi, val, -3.4e38))
        return jnp.where(lanes == l_splat, value, lc)

    def slow_path(xv, out_m, lc0):
        def cond(carry):
            in_m, _ = carry
            return plsc.all_reduce_population_count(in_m)[0] > 0

        def body(carry):
            in_m, lc = carry
            l_splat = plsc.all_reduce_ffs(in_m)
            lc = brute_lane(xv, l_splat, lc)
            return in_m & (lanes != l_splat), lc

        _, lc = lax.while_loop(cond, body, (~out_m, lc0))
        return lc

    with jax.named_scope("search_phase"):
        vregs_per_row = _D // _L

        @plsc.parallel_loop(0, _EPW // _L, unroll=2,
                            carry=jnp.zeros((_L,), jnp.float32))
        def acc(v, a):
            r = v // vregs_per_row
            col = (v % vregs_per_row) * _L
            xv = x_v[r, pl.ds(col, _L)]
            lo_m = xv <= smin
            hi_m = xv >= smax
            out_m = lo_m | hi_m
            n_out = plsc.all_reduce_population_count(out_m)
            lc0 = jnp.where(hi_m, smax, smin)
            lc = lax.cond(n_out[0] == _L,
                          lambda: lc0,
                          lambda: slow_path(xv, out_m, lc0))
            o_v[r, pl.ds(col, _L)] = xv + (lc - xv)
            d = lc - xv
            return a + d * d

    acc_v[...] = acc
    pltpu.sync_copy(o_v, out_hbm.at[pl.ds(row0, _ROWS)])
    pltpu.sync_copy(acc_v, sq_hbm.at[wid])


def kernel(encoded, embeddings):
    latent_code_st, sq = _vq_snap(encoded, embeddings)
    loss = 2.0 * (jnp.sum(sq) / encoded.shape[0])
    return latent_code_st, loss


# speedup vs baseline: 3.8851x; 1.0844x over previous
import functools

import jax
import jax.numpy as jnp
from jax import lax
from jax.experimental import pallas as pl
from jax.experimental.pallas import tpu as pltpu
from jax.experimental.pallas import tpu_sc as plsc

_B = 128
_D = 512
_N = _B * _D
_K = 2048
_NC = 2
_NS = 16
_L = 16
_NW = _NC * _NS
_EPW = _N // _NW
_ROWS = _EPW // _D


@functools.partial(
    pl.kernel,
    out_type=(
        jax.ShapeDtypeStruct((_B, _D), jnp.float32),
        jax.ShapeDtypeStruct((_NW, _L), jnp.float32),
    ),
    mesh=plsc.VectorSubcoreMesh(core_axis_name="c", subcore_axis_name="s",
                                num_cores=_NC, num_subcores=_NS),
    compiler_params=pltpu.CompilerParams(needs_layout_passes=False),
    scratch_types=[
        pltpu.VMEM((_K,), jnp.float32),
        pltpu.VMEM((_ROWS, _D), jnp.float32),
        pltpu.VMEM((_ROWS, _D), jnp.float32),
        pltpu.VMEM((_L,), jnp.float32),
    ],
)
def _vq_snap(x_hbm, emb_hbm, out_hbm, sq_hbm, emb_v, x_v, o_v, acc_v):
    cid = lax.axis_index("c")
    sid = lax.axis_index("s")
    wid = sid * _NC + cid
    lanes = lax.iota(jnp.int32, _L)

    pltpu.sync_copy(emb_hbm.at[0], emb_v)
    row0 = wid * _ROWS
    with jax.named_scope("x_dma"):
        pltpu.sync_copy(x_hbm.at[pl.ds(row0, _ROWS)], x_v)

    with jax.named_scope("minmax"):
        def mm(jv, carry):
            lo, hi = carry
            cjv = emb_v[pl.ds(jv * _L, _L)]
            return jnp.minimum(lo, cjv), jnp.maximum(hi, cjv)

        lo, hi = lax.fori_loop(
            0, _K // _L, mm,
            (jnp.full((_L,), jnp.inf, jnp.float32),
             jnp.full((_L,), -jnp.inf, jnp.float32)))
        smin = jnp.full((_L,), jnp.min(lo))
        smax = jnp.full((_L,), jnp.max(hi))

    def brute_lane(xv, l_splat, lc):
        xb = jnp.take_along_axis(xv, l_splat, axis=0)

        def scan_codes(jv, carry):
            dmin, val, idx = carry
            cjv = emb_v[pl.ds(jv * _L, _L)]
            d = (cjv - xb) * (cjv - xb)
            p = d < dmin
            jvec = jv * _L + lanes
            return (jnp.where(p, d, dmin), jnp.where(p, cjv, val),
                    jnp.where(p, jvec, idx))

        big = jnp.full((_L,), 3.4e38, jnp.float32)
        dmin, val, idx = lax.fori_loop(
            0, _K // _L, scan_codes,
            (big, jnp.zeros((_L,), jnp.float32),
             jnp.zeros((_L,), jnp.int32)))
        g = jnp.min(dmin)
        cand = jnp.where(dmin == g, idx, _K)
        bi = jnp.min(cand)
        value = jnp.max(jnp.where(cand == bi, val, -3.4e38))
        return jnp.where(lanes == l_splat, value, lc)

    def slow_path(xv, out_m, lc0):
        def cond(carry):
            in_m, _ = carry
            return plsc.all_reduce_population_count(in_m)[0] > 0

        def body(carry):
            in_m, lc = carry
            l_splat = plsc.all_reduce_ffs(in_m)
            lc = brute_lane(xv, l_splat, lc)
            return in_m & (lanes != l_splat), lc

        _, lc = lax.while_loop(cond, body, (~out_m, lc0))
        return lc

    with jax.named_scope("search_phase"):
        vregs_per_row = _D // _L
        _G = 4

        @plsc.parallel_loop(0, _EPW // _L // _G, unroll=2,
                            carry=jnp.zeros((_L,), jnp.float32))
        def acc(g, a):
            r = g // (vregs_per_row // _G)
            col0 = (g % (vregs_per_row // _G)) * (_G * _L)
            xs, his, outs = [], [], []
            all_out = None
            for u in range(_G):
                xv = x_v[r, pl.ds(col0 + u * _L, _L)]
                hi_m = xv >= smax
                out_m = (xv <= smin) | hi_m
                xs.append(xv)
                his.append(hi_m)
                outs.append(out_m)
                all_out = out_m if all_out is None else (all_out & out_m)
            n_out = plsc.all_reduce_population_count(all_out)

            def fast():
                return tuple(jnp.where(h, smax, smin) for h in his)

            def slow():
                lcs = []
                for u in range(_G):
                    lc0 = jnp.where(his[u], smax, smin)
                    n_u = plsc.all_reduce_population_count(outs[u])
                    lcs.append(lax.cond(
                        n_u[0] == _L,
                        lambda lc0=lc0: lc0,
                        lambda u=u, lc0=lc0: slow_path(xs[u], outs[u], lc0)))
                return tuple(lcs)

            lcs = lax.cond(n_out[0] == _L, fast, slow)
            for u in range(_G):
                xv = xs[u]
                lc = lcs[u]
                o_v[r, pl.ds(col0 + u * _L, _L)] = xv + (lc - xv)
                d = lc - xv
                a = a + d * d
            return a

    acc_v[...] = acc
    pltpu.sync_copy(o_v, out_hbm.at[pl.ds(row0, _ROWS)])
    pltpu.sync_copy(acc_v, sq_hbm.at[wid])


def kernel(encoded, embeddings):
    latent_code_st, sq = _vq_snap(encoded, embeddings)
    loss = 2.0 * (jnp.sum(sq) / encoded.shape[0])
    return latent_code_st, loss
